# Initial kernel scaffold; baseline (speedup 1.0000x reference)
import jax
import jax.numpy as jnp
from jax.experimental import pallas as pl


def kernel(payload, seq_lens):
    raise NotImplementedError("write your pallas kernel here")


# speedup vs baseline: 1.5319x; 1.5319x over previous
import functools

import jax
import jax.numpy as jnp
from jax import lax
from jax.experimental import pallas as pl
from jax.experimental.pallas import tpu as pltpu
from jax.experimental.pallas import tpu_sc as plsc

B, T, D = 16, 2048, 1024
L = 16
CHUNK = 64


@functools.cache
def _perm_const():
    rkeys = jax.random.split(jax.random.key(42), B)
    r = jax.vmap(lambda k: jax.random.uniform(k, (T,)))(rkeys)
    perm = jnp.argsort(r[:, 1:], axis=1, stable=True).astype(jnp.int32) + 1
    pad = jnp.full((B, 1), jnp.int32(1 << 30), jnp.int32)
    return jnp.concatenate([perm, pad], axis=1)


_mesh = plsc.VectorSubcoreMesh(core_axis_name="c", subcore_axis_name="s")


@functools.partial(
    pl.kernel,
    out_type=jax.ShapeDtypeStruct((B * T, D), jnp.float32),
    mesh=_mesh,
    compiler_params=pltpu.CompilerParams(needs_layout_passes=False),
    scratch_types=[
        pltpu.VMEM((T,), jnp.int32),
        pltpu.VMEM((T,), jnp.int32),
        pltpu.VMEM((L,), jnp.int32),
        pltpu.VMEM((CHUNK, D), jnp.float32),
        pltpu.SemaphoreType.DMA,
    ],
)
def _sc_shuffle(payload_hbm, slen_hbm, perm_hbm, out_hbm,
                idx_v, perm_v, slen_v, buf, sem):
    b = lax.axis_index("s")
    h = lax.axis_index("c")
    iota = lax.iota(jnp.int32, L)
    base = b * T

    pltpu.sync_copy(slen_hbm.at[b], slen_v)
    pltpu.sync_copy(perm_hbm.at[b], perm_v)
    slen = slen_v[...]

    def init_body(j, carry):
        idx_v[pl.ds(j * L, L)] = base + j * L + iota
        return carry

    lax.fori_loop(0, T // L, init_body, 0)

    def comp_body(j, off):
        pv = perm_v[pl.ds(j * L, L)]
        mask = pv <= slen
        mi = mask.astype(jnp.int32)
        dest = off + plsc.cumsum(mi) - 1
        plsc.store_scatter(idx_v, [dest], base + pv, mask=mask)
        return off + jnp.sum(mi)

    lax.fori_loop(0, T // L, comp_body, jnp.full((L,), 1, jnp.int32))

    half = h * (T // 2)
    for k in range((T // 2) // CHUNK):
        start = half + k * CHUNK
        pltpu.async_copy(
            payload_hbm.at[idx_v.at[pl.ds(start, CHUNK)]], buf, sem).wait()
        pltpu.sync_copy(buf, out_hbm.at[pl.ds(base + start, CHUNK)])


def kernel(payload, seq_lens):
    slen_splat = jnp.broadcast_to(
        seq_lens.astype(jnp.int32)[:, None], (B, L))
    out = _sc_shuffle(payload.reshape(B * T, D), slen_splat, _perm_const())
    return out.reshape(B, T, D)


# speedup vs baseline: 1.6505x; 1.0775x over previous
import functools

import jax
import jax.numpy as jnp
from jax import lax
from jax.experimental import pallas as pl
from jax.experimental.pallas import tpu as pltpu
from jax.experimental.pallas import tpu_sc as plsc

B, T, D = 16, 2048, 1024
L = 16
CHUNK = 32
NBUF = 3


@functools.cache
def _perm_const():
    rkeys = jax.random.split(jax.random.key(42), B)
    r = jax.vmap(lambda k: jax.random.uniform(k, (T,)))(rkeys)
    perm = jnp.argsort(r[:, 1:], axis=1, stable=True).astype(jnp.int32) + 1
    pad = jnp.full((B, 1), jnp.int32(1 << 30), jnp.int32)
    return jnp.concatenate([perm, pad], axis=1)


_mesh = plsc.VectorSubcoreMesh(core_axis_name="c", subcore_axis_name="s")


@functools.partial(
    pl.kernel,
    out_type=jax.ShapeDtypeStruct((B * T, D), jnp.float32),
    mesh=_mesh,
    compiler_params=pltpu.CompilerParams(needs_layout_passes=False),
    scratch_types=[
        pltpu.VMEM((T,), jnp.int32),
        pltpu.VMEM((T,), jnp.int32),
        pltpu.VMEM((L,), jnp.int32),
    ]
    + [pltpu.VMEM((CHUNK, D), jnp.float32) for _ in range(NBUF)]
    + [pltpu.SemaphoreType.DMA for _ in range(2 * NBUF)],
)
def _sc_shuffle(payload_hbm, slen_hbm, perm_hbm, out_hbm,
                idx_v, perm_v, slen_v, *bufs_sems):
    bufs = bufs_sems[:NBUF]
    gsems = bufs_sems[NBUF:2 * NBUF]
    osems = bufs_sems[2 * NBUF:]
    b = lax.axis_index("s")
    h = lax.axis_index("c")
    iota = lax.iota(jnp.int32, L)
    base = b * T

    pltpu.sync_copy(slen_hbm.at[b], slen_v)
    pltpu.sync_copy(perm_hbm.at[b], perm_v)
    slen = slen_v[...]

    def init_body(j, carry):
        idx_v[pl.ds(j * L, L)] = base + j * L + iota
        return carry

    lax.fori_loop(0, T // L, init_body, 0)

    def comp_body(j, off):
        pv = perm_v[pl.ds(j * L, L)]
        mask = pv <= slen
        mi = mask.astype(jnp.int32)
        dest = off + plsc.cumsum(mi) - 1
        plsc.store_scatter(idx_v, [dest], base + pv, mask=mask)
        return off + jnp.sum(mi)

    lax.fori_loop(0, T // L, comp_body, jnp.full((L,), 1, jnp.int32))

    half = h * (T // 2)
    nchunks = (T // 2) // CHUNK
    g_handles = [None] * NBUF
    o_handles = [None] * NBUF

    def start_gather(k):
        i = k % NBUF
        g_handles[i] = pltpu.async_copy(
            payload_hbm.at[idx_v.at[pl.ds(half + k * CHUNK, CHUNK)]],
            bufs[i], gsems[i])

    def drain_and_write(k):
        i = k % NBUF
        g_handles[i].wait()
        o_handles[i] = pltpu.async_copy(
            bufs[i], out_hbm.at[pl.ds(base + half + k * CHUNK, CHUNK)],
            osems[i])

    for k in range(nchunks):
        i = k % NBUF
        if k >= NBUF:
            o_handles[i].wait()
        start_gather(k)
        if k >= NBUF - 1:
            drain_and_write(k - (NBUF - 1))
    for k in range(max(nchunks - (NBUF - 1), 0), nchunks):
        drain_and_write(k)
    for k in range(max(nchunks - NBUF, 0), nchunks):
        o_handles[k % NBUF].wait()


def kernel(payload, seq_lens):
    slen_splat = jnp.broadcast_to(
        seq_lens.astype(jnp.int32)[:, None], (B, L))
    out = _sc_shuffle(payload.reshape(B * T, D), slen_splat, _perm_const())
    return out.reshape(B, T, D)


# speedup vs baseline: 1.6677x; 1.0104x over previous
import functools

import jax
import jax.numpy as jnp
from jax import lax
from jax.experimental import pallas as pl
from jax.experimental.pallas import tpu as pltpu
from jax.experimental.pallas import tpu_sc as plsc

B, T, D = 16, 2048, 1024
L = 16
CHUNK = 32
NBUF = 3


@functools.cache
def _perm_const():
    rkeys = jax.random.split(jax.random.key(42), B)
    r = jax.vmap(lambda k: jax.random.uniform(k, (T,)))(rkeys)
    perm = jnp.argsort(r[:, 1:], axis=1, stable=True).astype(jnp.int32) + 1
    pad = jnp.full((B, 1), jnp.int32(1 << 30), jnp.int32)
    return jnp.concatenate([perm, pad], axis=1)


_mesh = plsc.VectorSubcoreMesh(core_axis_name="c", subcore_axis_name="s")


@functools.partial(
    pl.kernel,
    out_type=jax.ShapeDtypeStruct((B * T, D), jnp.float32),
    mesh=_mesh,
    compiler_params=pltpu.CompilerParams(needs_layout_passes=False),
    scratch_types=[
        pltpu.VMEM((T + L,), jnp.int32),
        pltpu.VMEM((T,), jnp.int32),
        pltpu.VMEM((L,), jnp.int32),
    ]
    + [pltpu.VMEM((CHUNK, D), jnp.float32) for _ in range(NBUF)]
    + [pltpu.SemaphoreType.DMA for _ in range(2 * NBUF)],
)
def _sc_shuffle(payload_hbm, slen_hbm, perm_hbm, out_hbm,
                idx_v, perm_v, slen_v, *bufs_sems):
    bufs = bufs_sems[:NBUF]
    gsems = bufs_sems[NBUF:2 * NBUF]
    osems = bufs_sems[2 * NBUF:]
    b = lax.axis_index("s")
    h = lax.axis_index("c")
    iota = lax.iota(jnp.int32, L)
    base = b * T

    pltpu.sync_copy(slen_hbm.at[b], slen_v)
    pltpu.sync_copy(perm_hbm.at[b], perm_v)
    slen = slen_v[...]

    idx_v[pl.ds(0, L)] = base + iota
    UNROLL = 4

    def comp_body(u, off):
        for v in range(UNROLL):
            j = u * UNROLL + v
            nxt = j * L + L
            idx_v[pl.ds(nxt, L)] = base + nxt + iota
            pv = perm_v[pl.ds(j * L, L)]
            mask = pv <= slen
            mi = mask.astype(jnp.int32)
            dest = off + plsc.cumsum(mi) - 1
            plsc.store_scatter(idx_v, [dest], base + pv, mask=mask)
            off = off + jnp.sum(mi)
        return off

    lax.fori_loop(0, (T // L) // UNROLL, comp_body,
                  jnp.full((L,), 1, jnp.int32))

    half = h * (T // 2)
    nchunks = (T // 2) // CHUNK
    g_handles = [None] * NBUF
    o_handles = [None] * NBUF

    def start_gather(k):
        i = k % NBUF
        g_handles[i] = pltpu.async_copy(
            payload_hbm.at[idx_v.at[pl.ds(half + k * CHUNK, CHUNK)]],
            bufs[i], gsems[i])

    def drain_and_write(k):
        i = k % NBUF
        g_handles[i].wait()
        o_handles[i] = pltpu.async_copy(
            bufs[i], out_hbm.at[pl.ds(base + half + k * CHUNK, CHUNK)],
            osems[i])

    for k in range(nchunks):
        i = k % NBUF
        if k >= NBUF:
            o_handles[i].wait()
        start_gather(k)
        if k >= NBUF - 1:
            drain_and_write(k - (NBUF - 1))
    for k in range(max(nchunks - (NBUF - 1), 0), nchunks):
        drain_and_write(k)
    for k in range(max(nchunks - NBUF, 0), nchunks):
        o_handles[k % NBUF].wait()


def kernel(payload, seq_lens):
    slen_splat = jnp.broadcast_to(
        seq_lens.astype(jnp.int32)[:, None], (B, L))
    out = _sc_shuffle(payload.reshape(B * T, D), slen_splat, _perm_const())
    return out.reshape(B, T, D)


# speedup vs baseline: 1.7971x; 1.0776x over previous
import functools

import jax
import jax.numpy as jnp
import numpy as np
from jax import lax
from jax.experimental import pallas as pl
from jax.experimental.pallas import tpu as pltpu
from jax.experimental.pallas import tpu_sc as plsc

B, T, D = 16, 2048, 1024
L = 16
CHUNK = 32
NBUF = 3


@functools.cache
def _perm_const():
    with jax.ensure_compile_time_eval():
        rkeys = jax.random.split(jax.random.key(42), B)
        r = jax.vmap(lambda k: jax.random.uniform(k, (T,)))(rkeys)
        perm = jnp.argsort(r[:, 1:], axis=1, stable=True).astype(jnp.int32) + 1
        pad = jnp.full((B, 1), jnp.int32(1 << 30), jnp.int32)
        return np.asarray(jnp.concatenate([perm, pad], axis=1))


_mesh = plsc.VectorSubcoreMesh(core_axis_name="c", subcore_axis_name="s")


@functools.partial(
    pl.kernel,
    out_type=jax.ShapeDtypeStruct((B * T, D), jnp.float32),
    mesh=_mesh,
    compiler_params=pltpu.CompilerParams(needs_layout_passes=False),
    scratch_types=[
        pltpu.VMEM((T + L,), jnp.int32),
        pltpu.VMEM((T,), jnp.int32),
        pltpu.VMEM((L,), jnp.int32),
    ]
    + [pltpu.VMEM((CHUNK, D), jnp.float32) for _ in range(NBUF)]
    + [pltpu.SemaphoreType.DMA for _ in range(2 * NBUF)],
)
def _sc_shuffle(payload_hbm, slen_hbm, perm_hbm, out_hbm,
                idx_v, perm_v, slen_v, *bufs_sems):
    bufs = bufs_sems[:NBUF]
    gsems = bufs_sems[NBUF:2 * NBUF]
    osems = bufs_sems[2 * NBUF:]
    b = lax.axis_index("s")
    h = lax.axis_index("c")
    iota = lax.iota(jnp.int32, L)
    base = b * T

    pltpu.sync_copy(slen_hbm.at[b], slen_v)
    pltpu.sync_copy(perm_hbm.at[b], perm_v)
    slen = slen_v[...]

    idx_v[pl.ds(0, L)] = base + iota
    UNROLL = 4

    def comp_body(u, off):
        for v in range(UNROLL):
            j = u * UNROLL + v
            nxt = j * L + L
            idx_v[pl.ds(nxt, L)] = base + nxt + iota
            pv = perm_v[pl.ds(j * L, L)]
            mask = pv <= slen
            mi = mask.astype(jnp.int32)
            dest = off + plsc.cumsum(mi) - 1
            plsc.store_scatter(idx_v, [dest], base + pv, mask=mask)
            off = off + jnp.sum(mi)
        return off

    lax.fori_loop(0, (T // L) // UNROLL, comp_body,
                  jnp.full((L,), 1, jnp.int32))

    half = h * (T // 2)
    nchunks = (T // 2) // CHUNK
    g_handles = [None] * NBUF
    o_handles = [None] * NBUF

    def start_gather(k):
        i = k % NBUF
        g_handles[i] = pltpu.async_copy(
            payload_hbm.at[idx_v.at[pl.ds(half + k * CHUNK, CHUNK)]],
            bufs[i], gsems[i])

    def drain_and_write(k):
        i = k % NBUF
        g_handles[i].wait()
        o_handles[i] = pltpu.async_copy(
            bufs[i], out_hbm.at[pl.ds(base + half + k * CHUNK, CHUNK)],
            osems[i])

    for k in range(nchunks):
        i = k % NBUF
        if k >= NBUF:
            o_handles[i].wait()
        start_gather(k)
        if k >= NBUF - 1:
            drain_and_write(k - (NBUF - 1))
    for k in range(max(nchunks - (NBUF - 1), 0), nchunks):
        drain_and_write(k)
    for k in range(max(nchunks - NBUF, 0), nchunks):
        o_handles[k % NBUF].wait()


def kernel(payload, seq_lens):
    slen_splat = jnp.broadcast_to(
        seq_lens.astype(jnp.int32)[:, None], (B, L))
    out = _sc_shuffle(payload.reshape(B * T, D), slen_splat, _perm_const())
    return out.reshape(B, T, D)
